# Initial kernel scaffold; baseline (speedup 1.0000x reference)
import jax
import jax.numpy as jnp
from jax.experimental import pallas as pl


def kernel(indices, table):
    raise NotImplementedError("write your pallas kernel here")


# speedup vs baseline: 1.1049x; 1.1049x over previous
import functools

import jax
import jax.numpy as jnp
from jax import lax
from jax.experimental import pallas as pl
from jax.experimental.pallas import tpu as pltpu
from jax.experimental.pallas import tpu_sc as plsc

NC, NS = 2, 16
NW = NC * NS
CH = 128
K = 10
ROWS_G = CH * K
NG = 20
PW = ROWS_G * NG
B = PW * NW
D = 32

_mesh = plsc.VectorSubcoreMesh(core_axis_name="c", subcore_axis_name="s")


@functools.partial(
    pl.kernel,
    out_type=jax.ShapeDtypeStruct((B, D), jnp.float32),
    mesh=_mesh,
    scratch_types=[
        pltpu.VMEM((NG * K, CH), jnp.int32),
        pltpu.VMEM((ROWS_G, D), jnp.float32),
        pltpu.SemaphoreType.DMA,
    ],
    compiler_params=pltpu.CompilerParams(use_tc_tiling_on_sc=False),
)
def _lookup(idx_hbm, table_hbm, out_hbm, idx_v, buf, gsem):
    wid = lax.axis_index("s") * NC + lax.axis_index("c")
    pltpu.sync_copy(idx_hbm.at[wid], idx_v)
    base = wid * PW

    def group(g, carry):
        copies = []
        for k in range(K):
            copies.append(pltpu.async_copy(
                table_hbm.at[idx_v.at[g * K + k]],
                buf.at[pl.ds(k * CH, CH)],
                gsem,
            ))
        for cp in copies:
            cp.wait()
        pltpu.sync_copy(buf, out_hbm.at[pl.ds(base + g * ROWS_G, ROWS_G)])
        return carry

    lax.fori_loop(0, NG, group, 0)


def kernel(indices, table):
    idx = indices.astype(jnp.int32).reshape(NW, NG * K, CH)
    out = _lookup(idx, table)
    return out.reshape(indices.shape[0], indices.shape[1], D)


# speedup vs baseline: 1.1129x; 1.0072x over previous
import functools

import jax
import jax.numpy as jnp
from jax import lax
from jax.experimental import pallas as pl
from jax.experimental.pallas import tpu as pltpu
from jax.experimental.pallas import tpu_sc as plsc

NC, NS = 2, 16
NW = NC * NS
CH = 128
K = 5
ROWS_G = CH * K
R = 4
ROUNDS = 10
NG = R * ROUNDS
PW = ROWS_G * NG
B = PW * NW
D = 32

_mesh = plsc.VectorSubcoreMesh(core_axis_name="c", subcore_axis_name="s")


@functools.partial(
    pl.kernel,
    out_type=jax.ShapeDtypeStruct((B, D), jnp.float32),
    mesh=_mesh,
    scratch_types=[
        pltpu.VMEM((NG * K, CH), jnp.int32),
        [pltpu.VMEM((ROWS_G, D), jnp.float32) for _ in range(R)],
        [pltpu.SemaphoreType.DMA for _ in range(R)],
        [pltpu.SemaphoreType.DMA for _ in range(R)],
    ],
    compiler_params=pltpu.CompilerParams(use_tc_tiling_on_sc=False),
)
def _lookup(idx_hbm, table_hbm, out_hbm, idx_v, bufs, gsems, wsems):
    wid = lax.axis_index("s") * NC + lax.axis_index("c")
    pltpu.sync_copy(idx_hbm.at[wid], idx_v)
    base = wid * PW

    def gather_cp(g, slot, k):
        return pltpu.make_async_copy(
            table_hbm.at[idx_v.at[g * K + k]],
            bufs[slot].at[pl.ds(k * CH, CH)],
            gsems[slot],
        )

    def write_cp(g, slot):
        return pltpu.make_async_copy(
            bufs[slot], out_hbm.at[pl.ds(base + g * ROWS_G, ROWS_G)],
            wsems[slot],
        )

    for r in range(R - 1):
        for k in range(K):
            gather_cp(r, r, k).start()

    def round_body(rnd, carry):
        for r in range(R):
            g = rnd * R + r
            slot_m = (r - 1) % R

            def refill():
                m = g + R - 1
                write_cp(g - 1, slot_m).wait()
                for k in range(K):
                    gather_cp(m, slot_m, k).start()

            if r == 0:
                pl.when(rnd > 0)(lambda: write_cp(g - 1, slot_m).wait())
                for k in range(K):
                    gather_cp(g + R - 1, slot_m, k).start()
            else:
                pl.when(rnd < ROUNDS - 1)(refill)

            for k in range(K):
                gather_cp(g, r, k).wait()
            write_cp(g, r).start()
        return carry

    lax.fori_loop(0, ROUNDS, round_body, 0)

    for r in range(R):
        write_cp(NG - R + r, r).wait()


def kernel(indices, table):
    idx = indices.astype(jnp.int32).reshape(NW, NG * K, CH)
    out = _lookup(idx, table)
    return out.reshape(indices.shape[0], indices.shape[1], D)


# speedup vs baseline: 1.8044x; 1.6214x over previous
import functools

import jax
import jax.numpy as jnp
from jax import lax
from jax.experimental import pallas as pl
from jax.experimental.pallas import tpu as pltpu
from jax.experimental.pallas import tpu_sc as plsc

NC, NS = 2, 16
NW = NC * NS
BATCH = 16384
HIST = 50
D = 32
PB = BATCH // NW
G = 8
R = 4
NG = PB // G
ROUNDS = NG // R

_mesh = plsc.VectorSubcoreMesh(core_axis_name="c", subcore_axis_name="s")


@functools.partial(
    pl.kernel,
    out_type=jax.ShapeDtypeStruct((BATCH, HIST, D), jnp.float32),
    mesh=_mesh,
    scratch_types=[
        pltpu.VMEM((PB, HIST), jnp.int32),
        [pltpu.VMEM((G, HIST, D), jnp.float32) for _ in range(R)],
        [pltpu.SemaphoreType.DMA for _ in range(R)],
        [pltpu.SemaphoreType.DMA for _ in range(R)],
    ],
    compiler_params=pltpu.CompilerParams(use_tc_tiling_on_sc=False),
)
def _lookup(idx_hbm, table_hbm, out_hbm, idx_v, bufs, gsems, wsems):
    wid = lax.axis_index("s") * NC + lax.axis_index("c")
    b0 = wid * PB
    pltpu.sync_copy(idx_hbm.at[pl.ds(b0, PB)], idx_v)

    def gather_cps(g, slot):
        return [
            pltpu.make_async_copy(
                table_hbm.at[idx_v.at[g * G + k]],
                bufs[slot].at[k],
                gsems[slot],
            )
            for k in range(G)
        ]

    def start_gathers(g, slot):
        for cp in gather_cps(g, slot):
            cp.start()

    def wait_gathers(g, slot):
        for cp in gather_cps(g, slot):
            cp.wait()

    def write_cp(g, slot):
        return pltpu.make_async_copy(
            bufs[slot], out_hbm.at[pl.ds(b0 + g * G, G)],
            wsems[slot],
        )

    for r in range(R - 1):
        start_gathers(r, r)

    def round_body(rnd, carry):
        for r in range(R):
            g = rnd * R + r
            slot_m = (r - 1) % R

            def refill():
                write_cp(g - 1, slot_m).wait()
                start_gathers(g + R - 1, slot_m)

            if r == 0:
                pl.when(rnd > 0)(lambda: write_cp(g - 1, slot_m).wait())
                start_gathers(g + R - 1, slot_m)
            else:
                pl.when(rnd < ROUNDS - 1)(refill)

            wait_gathers(g, r)
            write_cp(g, r).start()
        return carry

    lax.fori_loop(0, ROUNDS, round_body, 0)

    for r in range(R):
        write_cp(NG - R + r, r).wait()


def kernel(indices, table):
    return _lookup(indices.astype(jnp.int32), table)
